# Initial kernel scaffold; baseline (speedup 1.0000x reference)
import jax
import jax.numpy as jnp
from jax.experimental import pallas as pl


def kernel(x, pos, edge_index, W_lin, b_lin, W_glob):
    raise NotImplementedError("write your pallas kernel here")


# speedup vs baseline: 8.8421x; 8.8421x over previous
import functools
import jax
import jax.numpy as jnp
from jax import lax
from jax.experimental import pallas as pl
from jax.experimental.pallas import tpu as pltpu
from jax.experimental.pallas import tpu_sc as plsc

N = 10000
E = 320000
D = 128

NC = 2
NS = 16
VEC = 16
RPT = 320
N_PAD = NC * NS * RPT
ES = E // NS
CHUNK = 2000
CAPB = 832
CAPT = NS * CAPB
GB = 96
ACC_ROWS = 328
RCNT_LEN = 352
DUMROW_CNT = 336
PACK_SHIFT = 18
PACK_MASK = (1 << PACK_SHIFT) - 1


def _tc_pre_body(x_ref, pos_ref, w_ref, b_ref, z_ref, p_ref):
    xx = x_ref[...]
    pos2 = pos_ref[:, :2]
    wx = w_ref[:, :D]
    wp = w_ref[:, D:D + 2]
    p = jnp.dot(pos2, wp.T, preferred_element_type=jnp.float32)
    z = jnp.dot(xx, wx.T, preferred_element_type=jnp.float32) + p + b_ref[...]
    z_ref[...] = z
    p_ref[...] = p


def _tc_post_body(seg_ref, p_ref, wg_ref, out_ref):
    s = seg_ref[:N, :] - p_ref[...]
    out_ref[...] = jnp.dot(s, wg_ref[...].T, preferred_element_type=jnp.float32)


def _sc_body(z_hbm, dst_hbm, src_hbm, seg_hbm,
             dchunk, schunk, pool_v, cnt_v, cnts_all, rcnt_v,
             ssrc_v, srow_v, acc_v,
             idx0, idx1, rows0, rows1,
             pool_sh, cnt_sh,
             sem0, sem1):
    c = lax.axis_index("c")
    s = lax.axis_index("s")
    iota = lax.iota(jnp.int32, VEC)
    zero16 = jnp.zeros((VEC,), jnp.int32)
    neg = jnp.full((VEC,), -jnp.inf, jnp.float32)

    def init_row(r, _):
        for j in range(D // VEC):
            acc_v[r, pl.ds(j * VEC, VEC)] = neg
        return 0

    lax.fori_loop(0, ACC_ROWS, init_row, 0)
    cnt_v[pl.ds(0, VEC)] = zero16
    cnt_v[pl.ds(VEC, VEC)] = zero16

    def init_rcnt(v, _):
        rcnt_v[pl.ds(v * VEC, VEC)] = zero16
        return 0

    lax.fori_loop(0, RCNT_LEN // VEC, init_rcnt, 0)

    cbase = c * NS

    def scan_chunk(ci, _):
        base = s * ES + ci * CHUNK
        pltpu.sync_copy(dst_hbm.at[pl.ds(base, CHUNK)], dchunk)
        pltpu.sync_copy(src_hbm.at[pl.ds(base, CHUNK)], schunk)

        def scan_vec(vi, _2):
            d = dchunk[pl.ds(vi * VEC, VEC)]
            sv = schunk[pl.ds(vi * VEC, VEC)]
            ow = jnp.right_shift(d * 3277, 20)
            ol = ow - cbase
            m = (ol >= 0) & (ol < NS)
            o = jnp.where(m, ol, NS)
            rank, last = plsc.scan_count(o)
            cnts = plsc.load_gather(cnt_v, [o])
            pos = o * CAPB + cnts + (rank - 1)
            lrow = d - ow * RPT
            packed = jnp.left_shift(lrow, PACK_SHIFT) | sv
            plsc.store_scatter(pool_v, [pos], packed, mask=m)
            plsc.store_scatter(cnt_v, [o], cnts + rank, mask=last)
            return 0

        lax.fori_loop(0, CHUNK // VEC, scan_vec, 0)
        return 0

    lax.fori_loop(0, ES // CHUNK, scan_chunk, 0)

    pltpu.sync_copy(pool_v, pool_sh.at[pl.ds(s * CAPT, CAPT)])
    pltpu.sync_copy(cnt_v, cnt_sh.at[pl.ds(s * 2 * VEC, 2 * VEC)])
    plsc.subcore_barrier()

    pltpu.sync_copy(cnt_sh, cnts_all)
    svec = jnp.broadcast_to(s, (VEC,)).astype(jnp.int32)
    wcnts = plsc.load_gather(cnts_all, [iota * (2 * VEC) + svec])
    wtot = plsc.cumsum(wcnts)
    total = wtot[15]

    for w in range(NS):
        pltpu.sync_copy(pool_sh.at[pl.ds(w * CAPT + s * CAPB, CAPB)],
                        pool_v.at[pl.ds(w * CAPB, CAPB)])

    def seg_pass(body_fn):
        for w in range(NS):
            nw = wcnts[w]
            nv = (nw + VEC - 1) // VEC

            def vbody(v, _):
                pv = pool_v[pl.ds(w * CAPB + v * VEC, VEC)]
                valid = (v * VEC + iota) < jnp.broadcast_to(nw, (VEC,))
                body_fn(w, v, pv, valid)
                return 0

            lax.fori_loop(0, nv, vbody, 0)

    def count_body(w, v, pv, valid):
        lrow = jnp.right_shift(pv, PACK_SHIFT)
        lsel = jnp.where(valid, lrow, DUMROW_CNT)
        rank, last = plsc.scan_count(lsel)
        cg = plsc.load_gather(rcnt_v, [lsel])
        plsc.store_scatter(rcnt_v, [lsel], cg + rank, mask=last)

    seg_pass(count_body)

    def psum(v, run):
        x = rcnt_v[pl.ds(v * VEC, VEC)]
        incl = plsc.cumsum(x)
        rcnt_v[pl.ds(v * VEC, VEC)] = run + incl - x
        return run + jnp.broadcast_to(incl[15], (VEC,))

    lax.fori_loop(0, RCNT_LEN // VEC, psum, zero16)

    def scatter_body(w, v, pv, valid):
        lrow = jnp.right_shift(pv, PACK_SHIFT)
        lsel = jnp.where(valid, lrow, DUMROW_CNT)
        rank, last = plsc.scan_count(lsel)
        pstart = plsc.load_gather(rcnt_v, [lsel])
        pos = pstart + rank - 1
        plsc.store_scatter(ssrc_v, [pos], pv & PACK_MASK, mask=valid)
        plsc.store_scatter(srow_v, [pos], lsel, mask=valid)
        plsc.store_scatter(rcnt_v, [lsel], pstart + rank, mask=last)

    seg_pass(scatter_body)

    tsplat = jnp.broadcast_to(total, (VEC,))
    pad_row = jnp.full((VEC,), RPT, jnp.int32)
    for k in range(GB // VEC):
        padi = tsplat + k * VEC + iota
        plsc.store_scatter(srow_v, [padi], pad_row)
        plsc.store_scatter(ssrc_v, [padi], zero16)

    nb = (total + GB - 1) // GB

    def build_idx(idx_ref, b):
        for j in range(GB // VEC):
            idx_ref[pl.ds(j * VEC, VEC)] = ssrc_v[pl.ds(b * GB + j * VEC, VEC)]

    @pl.when(nb > 0)
    def _():
        build_idx(idx0, 0)
        pltpu.async_copy(z_hbm.at[idx0], rows0, sem0)

    def process(b, carry, idx_cur, rows_cur, sem_cur, idx_nxt, rows_nxt, sem_nxt):
        @pl.when(b + 1 < nb)
        def _():
            build_idx(idx_nxt, b + 1)
            pltpu.async_copy(z_hbm.at[idx_nxt], rows_nxt, sem_nxt)

        pltpu.make_async_copy(z_hbm.at[idx_cur], rows_cur, sem_cur).wait()

        def group(g, gc):
            prev = gc[0]
            regs = list(gc[1:])
            lrowv = srow_v[pl.ds(b * GB + g * VEC, VEC)]
            for l in range(VEC):
                row = lrowv[l]
                mb = jnp.broadcast_to(row != prev, (VEC,))
                for j in range(D // VEC):
                    cs = pl.ds(j * VEC, VEC)
                    dv = rows_cur[g * VEC + l, cs]
                    regs[j] = jnp.where(mb, dv, jnp.maximum(regs[j], dv))
                    acc_v[row, cs] = regs[j]
                prev = row
            return (prev, *regs)

        return lax.fori_loop(0, GB // VEC, group, carry)

    def batch(b, carry):
        return lax.cond(
            b % 2 == 0,
            lambda cr: process(b, cr, idx0, rows0, sem0, idx1, rows1, sem1),
            lambda cr: process(b, cr, idx1, rows1, sem1, idx0, rows0, sem0),
            carry)

    init = (jnp.int32(-1),) + tuple(neg for _ in range(D // VEC))
    lax.fori_loop(0, nb, batch, init)

    gbase = (c * NS + s) * RPT
    pltpu.sync_copy(acc_v.at[pl.ds(0, RPT)], seg_hbm.at[pl.ds(gbase, RPT)])


@jax.jit
def kernel(x, pos, edge_index, W_lin, b_lin, W_glob):
    ei = edge_index.astype(jnp.int32)
    dst = ei[:, 0]
    src = ei[:, 1]

    z, p = pl.pallas_call(
        _tc_pre_body,
        out_shape=[jax.ShapeDtypeStruct((N, D), jnp.float32),
                   jax.ShapeDtypeStruct((N, D), jnp.float32)],
    )(x, pos, W_lin, b_lin.reshape(1, D))

    mesh = plsc.VectorSubcoreMesh(core_axis_name="c", subcore_axis_name="s")
    seg = pl.kernel(
        _sc_body,
        out_type=jax.ShapeDtypeStruct((N_PAD, D), jnp.float32),
        mesh=mesh,
        scratch_types=[
            pltpu.VMEM((CHUNK,), jnp.int32),
            pltpu.VMEM((CHUNK,), jnp.int32),
            pltpu.VMEM((CAPT,), jnp.int32),
            pltpu.VMEM((2 * VEC,), jnp.int32),
            pltpu.VMEM((NS * 2 * VEC,), jnp.int32),
            pltpu.VMEM((RCNT_LEN,), jnp.int32),
            pltpu.VMEM((CAPT + GB,), jnp.int32),
            pltpu.VMEM((CAPT + GB,), jnp.int32),
            pltpu.VMEM((ACC_ROWS, D), jnp.float32),
            pltpu.VMEM((GB,), jnp.int32),
            pltpu.VMEM((GB,), jnp.int32),
            pltpu.VMEM((GB, D), jnp.float32),
            pltpu.VMEM((GB, D), jnp.float32),
            pltpu.VMEM_SHARED((NS * CAPT,), jnp.int32),
            pltpu.VMEM_SHARED((NS * 2 * VEC,), jnp.int32),
            pltpu.SemaphoreType.DMA,
            pltpu.SemaphoreType.DMA,
        ],
        compiler_params=pltpu.CompilerParams(needs_layout_passes=False),
    )(z, dst, src)

    out = pl.pallas_call(
        _tc_post_body,
        out_shape=jax.ShapeDtypeStruct((N, D), jnp.float32),
    )(seg, p, W_glob)
    return out


# speedup vs baseline: 11.0112x; 1.2453x over previous
import functools
import jax
import jax.numpy as jnp
from jax import lax
from jax.experimental import pallas as pl
from jax.experimental.pallas import tpu as pltpu
from jax.experimental.pallas import tpu_sc as plsc

N = 10000
E = 320000
D = 128

NC = 2
NS = 16
VEC = 16
RPT = 320
N_PAD = NC * NS * RPT
ES = E // NS
CHUNK = 2000
CAPB = 832
CAPT = NS * CAPB
GB = 96
ACC_ROWS = 328
RCNT_LEN = 352
DUMROW_CNT = 336
PACK_SHIFT = 18
PACK_MASK = (1 << PACK_SHIFT) - 1


def _tc_pre_body(x_ref, pos_ref, w_ref, b_ref, z_ref, p_ref):
    xx = x_ref[...]
    pos2 = pos_ref[:, :2]
    wx = w_ref[:, :D]
    wp = w_ref[:, D:D + 2]
    p = jnp.dot(pos2, wp.T, preferred_element_type=jnp.float32)
    z = jnp.dot(xx, wx.T, preferred_element_type=jnp.float32) + p + b_ref[...]
    z_ref[...] = z
    p_ref[...] = p


def _tc_post_body(seg_ref, p_ref, wg_ref, out_ref):
    s = seg_ref[:N, :] - p_ref[...]
    out_ref[...] = jnp.dot(s, wg_ref[...].T, preferred_element_type=jnp.float32)


def _sc_body(z_hbm, dst_hbm, src_hbm, seg_hbm,
             dchunk, schunk, pool_v, cnt_v, cnts_all, rcnt_v,
             spack_v, acc_v,
             idx0, idx1, idx2, rows0, rows1, rows2,
             pool_sh, cnt_sh,
             sem0, sem1, sem2):
    c = lax.axis_index("c")
    s = lax.axis_index("s")
    iota = lax.iota(jnp.int32, VEC)
    zero16 = jnp.zeros((VEC,), jnp.int32)
    neg = jnp.full((VEC,), -jnp.inf, jnp.float32)

    def init_row(r, _):
        for j in range(D // VEC):
            acc_v[r, pl.ds(j * VEC, VEC)] = neg
        return 0

    lax.fori_loop(0, ACC_ROWS, init_row, 0)
    cnt_v[pl.ds(0, VEC)] = zero16
    cnt_v[pl.ds(VEC, VEC)] = zero16

    def init_rcnt(v, _):
        rcnt_v[pl.ds(v * VEC, VEC)] = zero16
        return 0

    lax.fori_loop(0, RCNT_LEN // VEC, init_rcnt, 0)

    cbase = c * NS

    def scan_chunk(ci, _):
        base = s * ES + ci * CHUNK
        pltpu.sync_copy(dst_hbm.at[pl.ds(base, CHUNK)], dchunk)
        pltpu.sync_copy(src_hbm.at[pl.ds(base, CHUNK)], schunk)

        def scan_vec(vi, _2):
            d = dchunk[pl.ds(vi * VEC, VEC)]
            sv = schunk[pl.ds(vi * VEC, VEC)]
            ow = jnp.right_shift(d * 3277, 20)
            ol = ow - cbase
            m = (ol >= 0) & (ol < NS)
            o = jnp.where(m, ol, NS)
            rank, last = plsc.scan_count(o)
            cnts = plsc.load_gather(cnt_v, [o])
            pos = o * CAPB + cnts + (rank - 1)
            lrow = d - ow * RPT
            packed = jnp.left_shift(lrow, PACK_SHIFT) | sv
            plsc.store_scatter(pool_v, [pos], packed, mask=m)
            plsc.store_scatter(cnt_v, [o], cnts + rank, mask=last)
            return 0

        lax.fori_loop(0, CHUNK // VEC, scan_vec, 0)
        return 0

    lax.fori_loop(0, ES // CHUNK, scan_chunk, 0)

    pltpu.sync_copy(pool_v, pool_sh.at[pl.ds(s * CAPT, CAPT)])
    pltpu.sync_copy(cnt_v, cnt_sh.at[pl.ds(s * 2 * VEC, 2 * VEC)])
    plsc.subcore_barrier()

    pltpu.sync_copy(cnt_sh, cnts_all)
    svec = jnp.broadcast_to(s, (VEC,)).astype(jnp.int32)
    wcnts = plsc.load_gather(cnts_all, [iota * (2 * VEC) + svec])
    wtot = plsc.cumsum(wcnts)
    total = wtot[15]

    for w in range(NS):
        pltpu.sync_copy(pool_sh.at[pl.ds(w * CAPT + s * CAPB, CAPB)],
                        pool_v.at[pl.ds(w * CAPB, CAPB)])

    def seg_pass(body_fn):
        for w in range(NS):
            nw = wcnts[w]
            nv = (nw + VEC - 1) // VEC

            def vbody(v, _):
                pv = pool_v[pl.ds(w * CAPB + v * VEC, VEC)]
                valid = (v * VEC + iota) < jnp.broadcast_to(nw, (VEC,))
                body_fn(w, v, pv, valid)
                return 0

            lax.fori_loop(0, nv, vbody, 0)

    def count_body(w, v, pv, valid):
        lrow = jnp.right_shift(pv, PACK_SHIFT)
        lsel = jnp.where(valid, lrow, DUMROW_CNT)
        rank, last = plsc.scan_count(lsel)
        cg = plsc.load_gather(rcnt_v, [lsel])
        plsc.store_scatter(rcnt_v, [lsel], cg + rank, mask=last)

    seg_pass(count_body)

    def psum(v, run):
        x = rcnt_v[pl.ds(v * VEC, VEC)]
        incl = plsc.cumsum(x)
        rcnt_v[pl.ds(v * VEC, VEC)] = run + incl - x
        return run + jnp.broadcast_to(incl[15], (VEC,))

    lax.fori_loop(0, RCNT_LEN // VEC, psum, zero16)

    def scatter_body(w, v, pv, valid):
        lrow = jnp.right_shift(pv, PACK_SHIFT)
        lsel = jnp.where(valid, lrow, DUMROW_CNT)
        rank, last = plsc.scan_count(lsel)
        pstart = plsc.load_gather(rcnt_v, [lsel])
        pos = pstart + rank - 1
        plsc.store_scatter(spack_v, [pos], pv, mask=valid)
        plsc.store_scatter(rcnt_v, [lsel], pstart + rank, mask=last)

    seg_pass(scatter_body)

    tsplat = jnp.broadcast_to(total, (VEC,))
    pad_val = jnp.full((VEC,), RPT << PACK_SHIFT, jnp.int32)
    for k in range(GB // VEC):
        padi = tsplat + k * VEC + iota
        plsc.store_scatter(spack_v, [padi], pad_val)

    nb = (total + GB - 1) // GB

    def build_idx(idx_ref, b):
        for j in range(GB // VEC):
            idx_ref[pl.ds(j * VEC, VEC)] = (
                spack_v[pl.ds(b * GB + j * VEC, VEC)] & PACK_MASK)

    @pl.when(nb > 0)
    def _():
        build_idx(idx0, 0)
        pltpu.async_copy(z_hbm.at[idx0], rows0, sem0)

    @pl.when(nb > 1)
    def _():
        build_idx(idx1, 1)
        pltpu.async_copy(z_hbm.at[idx1], rows1, sem1)

    def process(b, carry, idx_cur, rows_cur, sem_cur, idx_nxt, rows_nxt, sem_nxt):
        @pl.when(b + 2 < nb)
        def _():
            build_idx(idx_nxt, b + 2)
            pltpu.async_copy(z_hbm.at[idx_nxt], rows_nxt, sem_nxt)

        pltpu.make_async_copy(z_hbm.at[idx_cur], rows_cur, sem_cur).wait()

        def group(g, gc):
            lrowv = jnp.right_shift(spack_v[pl.ds(b * GB + g * VEC, VEC)],
                                    PACK_SHIFT)
            r0 = lrowv[0]
            uniform = jnp.all(lrowv == jnp.broadcast_to(r0, (VEC,)))

            def fast(fc):
                prev = fc[0]
                regs = list(fc[1:])
                mb = jnp.broadcast_to(r0 != prev, (VEC,))
                for j in range(D // VEC):
                    cs = pl.ds(j * VEC, VEC)
                    vals = [rows_cur[g * VEC + l, cs] for l in range(VEC)]
                    while len(vals) > 1:
                        vals = [jnp.maximum(vals[2 * i], vals[2 * i + 1])
                                for i in range(len(vals) // 2)]
                    t = vals[0]
                    regs[j] = jnp.where(mb, t, jnp.maximum(regs[j], t))
                    acc_v[r0, cs] = regs[j]
                return (r0, *regs)

            def slow(fc):
                prev = fc[0]
                regs = list(fc[1:])
                for l in range(VEC):
                    row = lrowv[l]
                    mb = jnp.broadcast_to(row != prev, (VEC,))
                    for j in range(D // VEC):
                        cs = pl.ds(j * VEC, VEC)
                        dv = rows_cur[g * VEC + l, cs]
                        regs[j] = jnp.where(mb, dv, jnp.maximum(regs[j], dv))
                        acc_v[row, cs] = regs[j]
                    prev = row
                return (prev, *regs)

            return lax.cond(uniform, fast, slow, gc)

        return lax.fori_loop(0, GB // VEC, group, carry)

    def batch(b, carry):
        r3 = b % 3
        return lax.cond(
            r3 == 0,
            lambda cr: process(b, cr, idx0, rows0, sem0, idx2, rows2, sem2),
            lambda cr: lax.cond(
                r3 == 1,
                lambda c2: process(b, c2, idx1, rows1, sem1, idx0, rows0, sem0),
                lambda c2: process(b, c2, idx2, rows2, sem2, idx1, rows1, sem1),
                cr),
            carry)

    init = (jnp.int32(-1),) + tuple(neg for _ in range(D // VEC))
    lax.fori_loop(0, nb, batch, init)

    gbase = (c * NS + s) * RPT
    pltpu.sync_copy(acc_v.at[pl.ds(0, RPT)], seg_hbm.at[pl.ds(gbase, RPT)])


@jax.jit
def kernel(x, pos, edge_index, W_lin, b_lin, W_glob):
    ei = edge_index.astype(jnp.int32)
    dst = ei[:, 0]
    src = ei[:, 1]

    z, p = pl.pallas_call(
        _tc_pre_body,
        out_shape=[jax.ShapeDtypeStruct((N, D), jnp.float32),
                   jax.ShapeDtypeStruct((N, D), jnp.float32)],
    )(x, pos, W_lin, b_lin.reshape(1, D))

    mesh = plsc.VectorSubcoreMesh(core_axis_name="c", subcore_axis_name="s")
    seg = pl.kernel(
        _sc_body,
        out_type=jax.ShapeDtypeStruct((N_PAD, D), jnp.float32),
        mesh=mesh,
        scratch_types=[
            pltpu.VMEM((CHUNK,), jnp.int32),
            pltpu.VMEM((CHUNK,), jnp.int32),
            pltpu.VMEM((CAPT,), jnp.int32),
            pltpu.VMEM((2 * VEC,), jnp.int32),
            pltpu.VMEM((NS * 2 * VEC,), jnp.int32),
            pltpu.VMEM((RCNT_LEN,), jnp.int32),
            pltpu.VMEM((CAPT + GB,), jnp.int32),
            pltpu.VMEM((ACC_ROWS, D), jnp.float32),
            pltpu.VMEM((GB,), jnp.int32),
            pltpu.VMEM((GB,), jnp.int32),
            pltpu.VMEM((GB,), jnp.int32),
            pltpu.VMEM((GB, D), jnp.float32),
            pltpu.VMEM((GB, D), jnp.float32),
            pltpu.VMEM((GB, D), jnp.float32),
            pltpu.VMEM_SHARED((NS * CAPT,), jnp.int32),
            pltpu.VMEM_SHARED((NS * 2 * VEC,), jnp.int32),
            pltpu.SemaphoreType.DMA,
            pltpu.SemaphoreType.DMA,
            pltpu.SemaphoreType.DMA,
        ],
        compiler_params=pltpu.CompilerParams(needs_layout_passes=False),
    )(z, dst, src)

    out = pl.pallas_call(
        _tc_post_body,
        out_shape=jax.ShapeDtypeStruct((N, D), jnp.float32),
    )(seg, p, W_glob)
    return out
